# baseline (speedup 1.0000x reference)
import functools

import jax
import jax.numpy as jnp
from jax import lax
from jax.experimental import pallas as pl
from jax.experimental.pallas import tpu as pltpu
from jax.experimental.pallas import tpu_sc as plsc

TOPK = 5
NROWS = 4096
NCOLS = 100000
LANES = 16
CHUNK = 128
NWORKERS = 32
ROWS_PER_W = NROWS // NWORKERS
BR = 256
BC = 4096
C0 = 65536
W_SC = NCOLS - C0
CB = 4096
T_FULL = W_SC // CB
TAIL = W_SC % CB


def _sc_gather_body(tbl_hbm, y_hbm, v_hbm, y_v, idx_v, chunks_v, sem):
    wid = lax.axis_index("s") * 2 + lax.axis_index("c")
    base = wid * ROWS_PER_W
    pltpu.sync_copy(y_hbm.at[pl.ds(base, ROWS_PER_W)], y_v)
    iota = lax.iota(jnp.int32, LANES)
    for g in range(ROWS_PER_W // LANES):
        yg = y_v[pl.ds(g * LANES, LANES)]
        rows = base + g * LANES + iota
        idx_v[pl.ds(g * LANES, LANES)] = (rows * NCOLS + yg) >> 7
    pltpu.async_copy(tbl_hbm.at[idx_v], chunks_v, sem).wait()
    pltpu.sync_copy(chunks_v, v_hbm.at[pl.ds(base, ROWS_PER_W)])


@functools.cache
def _sc_gather_kernel():
    return pl.kernel(
        _sc_gather_body,
        mesh=plsc.VectorSubcoreMesh(core_axis_name="c", subcore_axis_name="s"),
        out_type=jax.ShapeDtypeStruct((NROWS, CHUNK), jnp.float32),
        scratch_types=[
            pltpu.VMEM((ROWS_PER_W,), jnp.int32),
            pltpu.VMEM((ROWS_PER_W,), jnp.int32),
            pltpu.VMEM((ROWS_PER_W, CHUNK), jnp.float32),
            pltpu.SemaphoreType.DMA,
        ],
    )


def _sc_count_body(flat_hbm, y_hbm, v_hbm, cnt_hbm,
                   y_v, v_v, cnt_v, buf0, buf1, sem0, sem1):
    wid = lax.axis_index("s") * 2 + lax.axis_index("c")
    base = wid * ROWS_PER_W
    pltpu.sync_copy(y_hbm.at[pl.ds(base, ROWS_PER_W)], y_v)
    pltpu.sync_copy(v_hbm.at[pl.ds(base, ROWS_PER_W)], v_v)
    iota = lax.iota(jnp.int32, LANES)
    bufs = (buf0, buf1)
    sems = (sem0, sem1)
    sizes = [CB] * T_FULL + ([TAIL] if TAIL else [])

    def row_body(j, _):
        fb = (base + j) * NCOLS + C0
        y_b = y_v[j, pl.ds(0, LANES)]
        v_b = v_v[j, pl.ds(0, LANES)]

        def chunk(acc, t, cols0):
            def body2(k, carry):
                acc, cols = carry
                x = bufs[t % 2][pl.ds(k * LANES, LANES)]
                m = (x > v_b) | ((x == v_b) & (cols < y_b))
                acc = acc + jnp.where(m, 1, 0)
                return acc, cols + LANES
            acc, _ = lax.fori_loop(
                0, sizes[t] // LANES, body2, (acc, cols0 + iota))
            return acc

        cp = pltpu.async_copy(
            flat_hbm.at[pl.ds(fb, sizes[0])],
            buf0.at[pl.ds(0, sizes[0])], sem0)
        acc = jnp.zeros((LANES,), jnp.int32)
        off = 0
        for t in range(len(sizes)):
            if t + 1 < len(sizes):
                nxt = pltpu.async_copy(
                    flat_hbm.at[pl.ds(fb + off + sizes[t], sizes[t + 1])],
                    bufs[(t + 1) % 2].at[pl.ds(0, sizes[t + 1])],
                    sems[(t + 1) % 2])
            cp.wait()
            acc = chunk(acc, t, C0 + off)
            off += sizes[t]
            if t + 1 < len(sizes):
                cp = nxt
        cnt_v[pl.ds(j * LANES, LANES)] = acc
        return 0

    lax.fori_loop(0, ROWS_PER_W, row_body, 0)
    pltpu.sync_copy(cnt_v, cnt_hbm.at[wid])


@functools.cache
def _sc_count_kernel():
    return pl.kernel(
        _sc_count_body,
        mesh=plsc.VectorSubcoreMesh(core_axis_name="c", subcore_axis_name="s"),
        out_type=jax.ShapeDtypeStruct((NWORKERS, ROWS_PER_W * LANES), jnp.int32),
        scratch_types=[
            pltpu.VMEM((ROWS_PER_W, CHUNK), jnp.int32),
            pltpu.VMEM((ROWS_PER_W, CHUNK), jnp.float32),
            pltpu.VMEM((ROWS_PER_W * LANES,), jnp.int32),
            pltpu.VMEM((CB,), jnp.float32),
            pltpu.VMEM((CB,), jnp.float32),
            pltpu.SemaphoreType.DMA,
            pltpu.SemaphoreType.DMA,
        ],
    )


def _tc_extract_body(c_ref, y_ref, vf_ref, yf_ref):
    yv = y_ref[...]
    rows = lax.broadcasted_iota(jnp.int32, (NROWS, 1), 0)
    off = (rows * (NCOLS % CHUNK) + yv) & (CHUNK - 1)
    lane = lax.broadcasted_iota(jnp.int32, (NROWS, CHUNK), 1)
    v = jnp.where(lane == off, c_ref[...], 0.0).sum(axis=1, keepdims=True)
    vf_ref[...] = jnp.broadcast_to(v, (NROWS, CHUNK))
    yf_ref[...] = jnp.broadcast_to(yv, (NROWS, CHUNK))


def _tc_count_body(x_ref, v_ref, y_ref, out_ref, acc_ref):
    j = pl.program_id(1)

    @pl.when(j == 0)
    def _init():
        acc_ref[...] = jnp.zeros_like(acc_ref)

    x = x_ref[...]
    v = v_ref[:, pl.ds(0, 1)]
    yv = y_ref[:, pl.ds(0, 1)]
    cols = j * BC + lax.broadcasted_iota(jnp.int32, (BR, BC), 1)
    m = (x > v) | ((x == v) & (cols < yv))
    ones = jnp.where(m, 1.0, 0.0)
    acc_ref[...] += ones.reshape(BR, BC // 128, 128).sum(axis=1)

    @pl.when(j == pl.num_programs(1) - 1)
    def _fin():
        rank = acc_ref[...].sum(axis=1)
        out_ref[...] = rank.reshape(1, BR // CHUNK, CHUNK)


def _combine_body(rtc_ref, csc_ref, out_ref):
    r = rtc_ref[...].reshape(NWORKERS, ROWS_PER_W)
    c = csc_ref[...].reshape(NWORKERS, ROWS_PER_W, LANES).sum(axis=2)
    rank = r + c.astype(jnp.float32)
    match = jnp.where(rank < (TOPK - 0.5), 1.0, 0.0)
    out_ref[...] = jnp.sum(match, keepdims=True).reshape(1, 1) * (1.0 / NROWS)


def kernel(y_hat_logits, y):
    y32 = y.astype(jnp.int32)
    tbl = y_hat_logits.reshape(NROWS * NCOLS // CHUNK, CHUNK)
    flat = y_hat_logits.reshape(NROWS * NCOLS)
    chunks = _sc_gather_kernel()(tbl, y32)
    vf, yf = pl.pallas_call(
        _tc_extract_body,
        in_specs=[
            pl.BlockSpec((NROWS, CHUNK), lambda: (0, 0)),
            pl.BlockSpec((NROWS, 1), lambda: (0, 0)),
        ],
        out_specs=[
            pl.BlockSpec((NROWS, CHUNK), lambda: (0, 0)),
            pl.BlockSpec((NROWS, CHUNK), lambda: (0, 0)),
        ],
        out_shape=[
            jax.ShapeDtypeStruct((NROWS, CHUNK), jnp.float32),
            jax.ShapeDtypeStruct((NROWS, CHUNK), jnp.int32),
        ],
    )(chunks, y32.reshape(NROWS, 1))
    cnt_sc = _sc_count_kernel()(flat, yf, vf)
    rank_tc = pl.pallas_call(
        _tc_count_body,
        grid=(NROWS // BR, C0 // BC),
        in_specs=[
            pl.BlockSpec((BR, BC), lambda i, j: (i, j)),
            pl.BlockSpec((BR, CHUNK), lambda i, j: (i, 0)),
            pl.BlockSpec((BR, CHUNK), lambda i, j: (i, 0)),
        ],
        out_specs=pl.BlockSpec((1, BR // CHUNK, CHUNK), lambda i, j: (i, 0, 0)),
        out_shape=jax.ShapeDtypeStruct(
            (NROWS // BR, BR // CHUNK, CHUNK), jnp.float32),
        scratch_shapes=[pltpu.VMEM((BR, 128), jnp.float32)],
    )(y_hat_logits, vf, yf)
    out = pl.pallas_call(
        _combine_body,
        in_specs=[
            pl.BlockSpec(
                (NROWS // BR, BR // CHUNK, CHUNK), lambda: (0, 0, 0)),
            pl.BlockSpec((NWORKERS, ROWS_PER_W * LANES), lambda: (0, 0)),
        ],
        out_specs=pl.BlockSpec((1, 1), lambda: (0, 0)),
        out_shape=jax.ShapeDtypeStruct((1, 1), jnp.float32),
    )(rank_tc, cnt_sc)
    return out[0, 0]


# speedup vs baseline: 2.4160x; 2.4160x over previous
import functools

import jax
import jax.numpy as jnp
from jax import lax
from jax.experimental import pallas as pl
from jax.experimental.pallas import tpu as pltpu
from jax.experimental.pallas import tpu_sc as plsc

TOPK = 5
NROWS = 4096
NCOLS = 100000
LANES = 16
CHUNK = 128
NWORKERS = 32
ROWS_PER_W = NROWS // NWORKERS
BR = 256
BC = 4096


BATCH = 64


def _sc_gather_body(x_hbm, y_hbm, v_hbm, y_v, tiles_v, out_v, sem):
    wid = lax.axis_index("s") * 2 + lax.axis_index("c")
    base = wid * ROWS_PER_W
    pltpu.sync_copy(y_hbm.at[pl.ds(base, ROWS_PER_W)],
                    y_v.at[pl.ds(0, ROWS_PER_W)])

    for b in range(ROWS_PER_W // BATCH):
        def issue(j2, _):
            y_s = y_v[pl.ds(b * BATCH + j2, LANES)][0]
            r0 = pl.multiple_of(base + b * BATCH + (j2 & ~7), 8)
            c0 = pl.multiple_of(y_s & ~(CHUNK - 1), CHUNK)
            pltpu.async_copy(x_hbm.at[pl.ds(r0, 8), pl.ds(c0, CHUNK)],
                             tiles_v.at[j2], sem)
            return 0

        def drain(j2, _):
            pltpu.make_async_copy(x_hbm.at[pl.ds(0, 8), pl.ds(0, CHUNK)],
                                  tiles_v.at[j2], sem).wait()
            return 0

        def extract(j2, _):
            s8 = j2 & 7
            for k in range(CHUNK // LANES):
                out_v[b * BATCH + j2, pl.ds(k * LANES, LANES)] = (
                    tiles_v[j2, s8, pl.ds(k * LANES, LANES)])
            return 0

        lax.fori_loop(0, BATCH, issue, 0)
        lax.fori_loop(0, BATCH, drain, 0)
        lax.fori_loop(0, BATCH, extract, 0)

    pltpu.sync_copy(out_v, v_hbm.at[pl.ds(base, ROWS_PER_W)])


@functools.cache
def _sc_gather_kernel():
    return pl.kernel(
        _sc_gather_body,
        mesh=plsc.VectorSubcoreMesh(core_axis_name="c", subcore_axis_name="s"),
        out_type=jax.ShapeDtypeStruct((NROWS, CHUNK), jnp.float32),
        scratch_types=[
            pltpu.VMEM((ROWS_PER_W + LANES,), jnp.int32),
            pltpu.VMEM((BATCH, 8, CHUNK), jnp.float32),
            pltpu.VMEM((ROWS_PER_W, CHUNK), jnp.float32),
            pltpu.SemaphoreType.DMA,
        ],
    )


def _tc_extract_body(c_ref, y_ref, vf_ref, yf_ref):
    yv = y_ref[...]
    off = yv & (CHUNK - 1)
    lane = lax.broadcasted_iota(jnp.int32, (NROWS, CHUNK), 1)
    v = jnp.where(lane == off, c_ref[...], 0.0).sum(axis=1, keepdims=True)
    vf_ref[...] = jnp.broadcast_to(v, (NROWS, CHUNK))
    yf_ref[...] = jnp.broadcast_to(yv, (NROWS, CHUNK))


def _tc_count_body(x_ref, v_ref, y_ref, out_ref, acc_ref):
    j = pl.program_id(1)

    @pl.when(j == 0)
    def _init():
        acc_ref[...] = jnp.zeros_like(acc_ref)

    x = x_ref[...]
    v = v_ref[:, pl.ds(0, 1)]
    yv = y_ref[:, pl.ds(0, 1)]
    cols = j * BC + lax.broadcasted_iota(jnp.int32, (BR, BC), 1)
    m = (x > v) | ((x == v) & (cols < yv))
    m = m & (cols < NCOLS)
    ones = jnp.where(m, 1.0, 0.0)
    acc_ref[...] += ones.reshape(BR, BC // 128, 128).sum(axis=1)

    @pl.when(j == pl.num_programs(1) - 1)
    def _fin():
        rank = acc_ref[...].sum(axis=1, keepdims=True)
        match = jnp.where(rank < (TOPK - 0.5), 1.0, 0.0)
        part = jnp.sum(match, keepdims=True).reshape(1, 1)
        i = pl.program_id(0)
        prev = jnp.where(i == 0, jnp.zeros_like(part), out_ref[...])
        scale = jnp.where(i == pl.num_programs(0) - 1, 1.0 / NROWS, 1.0)
        out_ref[...] = (prev + part) * scale


def kernel(y_hat_logits, y):
    y32 = y.astype(jnp.int32)
    chunks = _sc_gather_kernel()(y_hat_logits, y32)
    vf, yf = pl.pallas_call(
        _tc_extract_body,
        in_specs=[
            pl.BlockSpec((NROWS, CHUNK), lambda: (0, 0)),
            pl.BlockSpec((NROWS, 1), lambda: (0, 0)),
        ],
        out_specs=[
            pl.BlockSpec((NROWS, CHUNK), lambda: (0, 0)),
            pl.BlockSpec((NROWS, CHUNK), lambda: (0, 0)),
        ],
        out_shape=[
            jax.ShapeDtypeStruct((NROWS, CHUNK), jnp.float32),
            jax.ShapeDtypeStruct((NROWS, CHUNK), jnp.int32),
        ],
    )(chunks, y32.reshape(NROWS, 1))
    out = pl.pallas_call(
        _tc_count_body,
        grid=(NROWS // BR, pl.cdiv(NCOLS, BC)),
        in_specs=[
            pl.BlockSpec((BR, BC), lambda i, j: (i, j)),
            pl.BlockSpec((BR, CHUNK), lambda i, j: (i, 0)),
            pl.BlockSpec((BR, CHUNK), lambda i, j: (i, 0)),
        ],
        out_specs=pl.BlockSpec((1, 1), lambda i, j: (0, 0)),
        out_shape=jax.ShapeDtypeStruct((1, 1), jnp.float32),
        scratch_shapes=[pltpu.VMEM((BR, 128), jnp.float32)],
    )(y_hat_logits, vf, yf)
    return out[0, 0]


# speedup vs baseline: 2.4171x; 1.0004x over previous
import functools

import jax
import jax.numpy as jnp
from jax import lax
from jax.experimental import pallas as pl
from jax.experimental.pallas import tpu as pltpu
from jax.experimental.pallas import tpu_sc as plsc

TOPK = 5
NROWS = 4096
NCOLS = 100000
LANES = 16
CHUNK = 128
NWORKERS = 32
ROWS_PER_W = NROWS // NWORKERS
BR = 256
BC = 4096


BATCH = 64


def _sc_gather_body(x_hbm, y_hbm, v_hbm, y_v, tiles_v, out_v, sem):
    wid = lax.axis_index("s") * 2 + lax.axis_index("c")
    base = wid * ROWS_PER_W
    pltpu.sync_copy(y_hbm.at[pl.ds(base, ROWS_PER_W)],
                    y_v.at[pl.ds(0, ROWS_PER_W)])

    for b in range(ROWS_PER_W // BATCH):
        def issue(j2, _):
            y_s = y_v[pl.ds(b * BATCH + j2, LANES)][0]
            r0 = pl.multiple_of(base + b * BATCH + (j2 & ~7), 8)
            c0 = pl.multiple_of(y_s & ~(CHUNK - 1), CHUNK)
            pltpu.async_copy(x_hbm.at[pl.ds(r0, 8), pl.ds(c0, CHUNK)],
                             tiles_v.at[j2], sem)
            return 0

        def drain(j2, _):
            pltpu.make_async_copy(x_hbm.at[pl.ds(0, 8), pl.ds(0, CHUNK)],
                                  tiles_v.at[j2], sem).wait()
            return 0

        def extract(j2, _):
            s8 = j2 & 7
            for k in range(CHUNK // LANES):
                out_v[b * BATCH + j2, pl.ds(k * LANES, LANES)] = (
                    tiles_v[j2, s8, pl.ds(k * LANES, LANES)])
            return 0

        lax.fori_loop(0, BATCH, issue, 0)
        lax.fori_loop(0, BATCH, drain, 0)
        lax.fori_loop(0, BATCH, extract, 0)

    pltpu.sync_copy(out_v, v_hbm.at[pl.ds(base, ROWS_PER_W)])


@functools.cache
def _sc_gather_kernel():
    return pl.kernel(
        _sc_gather_body,
        mesh=plsc.VectorSubcoreMesh(core_axis_name="c", subcore_axis_name="s"),
        compiler_params=pltpu.CompilerParams(use_tc_tiling_on_sc=True),
        out_type=jax.ShapeDtypeStruct((NROWS, CHUNK), jnp.float32),
        scratch_types=[
            pltpu.VMEM((ROWS_PER_W + LANES,), jnp.int32),
            pltpu.VMEM((BATCH, 8, CHUNK), jnp.float32),
            pltpu.VMEM((ROWS_PER_W, CHUNK), jnp.float32),
            pltpu.SemaphoreType.DMA,
        ],
    )


def _tc_extract_body(c_ref, y_ref, vf_ref, yf_ref):
    yv = y_ref[...]
    off = yv & (CHUNK - 1)
    lane = lax.broadcasted_iota(jnp.int32, (NROWS, CHUNK), 1)
    v = jnp.where(lane == off, c_ref[...], 0.0).sum(axis=1, keepdims=True)
    vf_ref[...] = jnp.broadcast_to(v, (NROWS, CHUNK))
    yf_ref[...] = jnp.broadcast_to(yv, (NROWS, CHUNK))


def _tc_count_body(x_ref, v_ref, y_ref, out_ref, acc_ref):
    j = pl.program_id(1)

    @pl.when(j == 0)
    def _init():
        acc_ref[...] = jnp.zeros_like(acc_ref)

    x = x_ref[...]
    v = v_ref[:, pl.ds(0, 1)]
    yv = y_ref[:, pl.ds(0, 1)]
    cols = j * BC + lax.broadcasted_iota(jnp.int32, (BR, BC), 1)
    m = (x > v) | ((x == v) & (cols < yv))
    m = m & (cols < NCOLS)
    ones = jnp.where(m, 1.0, 0.0)
    acc_ref[...] += ones.reshape(BR, BC // 128, 128).sum(axis=1)

    @pl.when(j == pl.num_programs(1) - 1)
    def _fin():
        rank = acc_ref[...].sum(axis=1, keepdims=True)
        match = jnp.where(rank < (TOPK - 0.5), 1.0, 0.0)
        part = jnp.sum(match, keepdims=True).reshape(1, 1)
        i = pl.program_id(0)
        prev = jnp.where(i == 0, jnp.zeros_like(part), out_ref[...])
        scale = jnp.where(i == pl.num_programs(0) - 1, 1.0 / NROWS, 1.0)
        out_ref[...] = (prev + part) * scale


def kernel(y_hat_logits, y):
    y32 = y.astype(jnp.int32)
    chunks = _sc_gather_kernel()(y_hat_logits, y32)
    vf, yf = pl.pallas_call(
        _tc_extract_body,
        in_specs=[
            pl.BlockSpec((NROWS, CHUNK), lambda: (0, 0)),
            pl.BlockSpec((NROWS, 1), lambda: (0, 0)),
        ],
        out_specs=[
            pl.BlockSpec((NROWS, CHUNK), lambda: (0, 0)),
            pl.BlockSpec((NROWS, CHUNK), lambda: (0, 0)),
        ],
        out_shape=[
            jax.ShapeDtypeStruct((NROWS, CHUNK), jnp.float32),
            jax.ShapeDtypeStruct((NROWS, CHUNK), jnp.int32),
        ],
    )(chunks, y32.reshape(NROWS, 1))
    out = pl.pallas_call(
        _tc_count_body,
        grid=(NROWS // BR, pl.cdiv(NCOLS, BC)),
        in_specs=[
            pl.BlockSpec((BR, BC), lambda i, j: (i, j)),
            pl.BlockSpec((BR, CHUNK), lambda i, j: (i, 0)),
            pl.BlockSpec((BR, CHUNK), lambda i, j: (i, 0)),
        ],
        out_specs=pl.BlockSpec((1, 1), lambda i, j: (0, 0)),
        out_shape=jax.ShapeDtypeStruct((1, 1), jnp.float32),
        scratch_shapes=[pltpu.VMEM((BR, 128), jnp.float32)],
    )(y_hat_logits, vf, yf)
    return out[0, 0]


# speedup vs baseline: 10.1457x; 4.1974x over previous
import functools

import jax
import jax.numpy as jnp
from jax import lax
from jax.experimental import pallas as pl
from jax.experimental.pallas import tpu as pltpu
from jax.experimental.pallas import tpu_sc as plsc

TOPK = 5
NROWS = 4096
NCOLS = 100000
LANES = 16
NWORKERS = 32
ROWS_PER_W = NROWS // NWORKERS
BATCH = 64
BCC = 4096
BRS = 512


def _sc_gather_body(xt_hbm, y_hbm, v_hbm, y_v, tiles_v, v_v, sem):
    wid = lax.axis_index("s") * 2 + lax.axis_index("c")
    base = pl.multiple_of(wid * ROWS_PER_W, ROWS_PER_W)
    pltpu.sync_copy(y_hbm.at[pl.ds(base, ROWS_PER_W)], y_v)
    iota = lax.iota(jnp.int32, LANES)
    for b in range(ROWS_PER_W // BATCH):
        for j in range(BATCH):
            lane = b * BATCH + j
            y_s = y_v[pl.ds((lane // LANES) * LANES, LANES)][lane % LANES]
            c8 = pl.multiple_of(y_s & ~7, 8)
            pltpu.async_copy(xt_hbm.at[pl.ds(c8, 8), pl.ds(base, ROWS_PER_W)],
                             tiles_v.at[j], sem)
        for j in range(BATCH):
            pltpu.make_async_copy(
                xt_hbm.at[pl.ds(0, 8), pl.ds(0, ROWS_PER_W)],
                tiles_v.at[j], sem).wait()
        for g in range(BATCH // LANES):
            grp = jnp.zeros((LANES,), jnp.float32)
            for k in range(LANES):
                j = g * LANES + k
                lane = b * BATCH + j
                y_s = y_v[pl.ds((lane // LANES) * LANES, LANES)][lane % LANES]
                s8 = y_s & 7
                vec = tiles_v[j, s8, pl.ds((lane // LANES) * LANES, LANES)]
                grp = jnp.where(iota == k, vec, grp)
            v_v[pl.ds(b * BATCH + g * LANES, LANES)] = grp
    pltpu.sync_copy(v_v, v_hbm.at[pl.ds(base, ROWS_PER_W)])


@functools.cache
def _sc_gather_kernel():
    return pl.kernel(
        _sc_gather_body,
        mesh=plsc.VectorSubcoreMesh(core_axis_name="c", subcore_axis_name="s"),
        out_type=jax.ShapeDtypeStruct((NROWS,), jnp.float32),
        scratch_types=[
            pltpu.VMEM((ROWS_PER_W,), jnp.int32),
            pltpu.VMEM((BATCH, 8, ROWS_PER_W), jnp.float32),
            pltpu.VMEM((ROWS_PER_W,), jnp.float32),
            pltpu.SemaphoreType.DMA,
        ],
    )


def _tc_count_body(x_ref, v_ref, y_ref, out_ref, acc_ref):
    j = pl.program_id(1)

    @pl.when(j == 0)
    def _init():
        acc_ref[...] = jnp.zeros_like(acc_ref)

    x = x_ref[...]
    v = v_ref[...]
    yv = y_ref[...]
    cols = j * BCC + lax.broadcasted_iota(jnp.int32, (BCC, BRS), 0)
    m = (x > v) | ((x == v) & (cols < yv))
    m = m & (cols < NCOLS)
    ones = jnp.where(m, 1.0, 0.0)
    acc_ref[...] += ones.reshape(BCC // 8, 8, BRS).sum(axis=0)

    @pl.when(j == pl.num_programs(1) - 1)
    def _fin():
        rank = acc_ref[...].sum(axis=0, keepdims=True)
        match = jnp.where(rank < (TOPK - 0.5), 1.0, 0.0)
        part = jnp.sum(match, keepdims=True).reshape(1, 1)
        i = pl.program_id(0)
        prev = jnp.where(i == 0, jnp.zeros_like(part), out_ref[...])
        scale = jnp.where(i == pl.num_programs(0) - 1, 1.0 / NROWS, 1.0)
        out_ref[...] = (prev + part) * scale


def kernel(y_hat_logits, y):
    y32 = y.astype(jnp.int32)
    xt = y_hat_logits.T
    v1 = _sc_gather_kernel()(xt, y32)
    out = pl.pallas_call(
        _tc_count_body,
        grid=(NROWS // BRS, pl.cdiv(NCOLS, BCC)),
        in_specs=[
            pl.BlockSpec((BCC, BRS), lambda i, j: (j, i)),
            pl.BlockSpec((1, BRS), lambda i, j: (0, i)),
            pl.BlockSpec((1, BRS), lambda i, j: (0, i)),
        ],
        out_specs=pl.BlockSpec((1, 1), lambda i, j: (0, 0)),
        out_shape=jax.ShapeDtypeStruct((1, 1), jnp.float32),
        scratch_shapes=[pltpu.VMEM((8, BRS), jnp.float32)],
    )(xt, v1.reshape(1, NROWS), y32.reshape(1, NROWS))
    return out[0, 0]


# speedup vs baseline: 11.9369x; 1.1766x over previous
import functools

import jax
import jax.numpy as jnp
from jax import lax
from jax.experimental import pallas as pl
from jax.experimental.pallas import tpu as pltpu
from jax.experimental.pallas import tpu_sc as plsc

TOPK = 5
NROWS = 4096
NCOLS = 100000
LANES = 16
NWORKERS = 32
ROWS_PER_W = NROWS // NWORKERS
BATCH = 64
BCC = 4000
BRS = 512


def _sc_gather_body(xt_hbm, y_hbm, v_hbm, y_v, tiles_v, v_v, sem):
    wid = lax.axis_index("s") * 2 + lax.axis_index("c")
    base = pl.multiple_of(wid * ROWS_PER_W, ROWS_PER_W)
    pltpu.sync_copy(y_hbm.at[pl.ds(base, ROWS_PER_W)], y_v)
    iota = lax.iota(jnp.int32, LANES)
    for b in range(ROWS_PER_W // BATCH):
        for j in range(BATCH):
            lane = b * BATCH + j
            y_s = y_v[pl.ds((lane // LANES) * LANES, LANES)][lane % LANES]
            c8 = pl.multiple_of(y_s & ~7, 8)
            pltpu.async_copy(xt_hbm.at[pl.ds(c8, 8), pl.ds(base, ROWS_PER_W)],
                             tiles_v.at[j], sem)
        for j in range(BATCH):
            pltpu.make_async_copy(
                xt_hbm.at[pl.ds(0, 8), pl.ds(0, ROWS_PER_W)],
                tiles_v.at[j], sem).wait()
        for g in range(BATCH // LANES):
            grp = jnp.zeros((LANES,), jnp.float32)
            for k in range(LANES):
                j = g * LANES + k
                lane = b * BATCH + j
                y_s = y_v[pl.ds((lane // LANES) * LANES, LANES)][lane % LANES]
                s8 = y_s & 7
                vec = tiles_v[j, s8, pl.ds((lane // LANES) * LANES, LANES)]
                grp = jnp.where(iota == k, vec, grp)
            v_v[pl.ds(b * BATCH + g * LANES, LANES)] = grp
    pltpu.sync_copy(v_v, v_hbm.at[pl.ds(base, ROWS_PER_W)])


@functools.cache
def _sc_gather_kernel():
    return pl.kernel(
        _sc_gather_body,
        mesh=plsc.VectorSubcoreMesh(core_axis_name="c", subcore_axis_name="s"),
        out_type=jax.ShapeDtypeStruct((NROWS,), jnp.float32),
        scratch_types=[
            pltpu.VMEM((ROWS_PER_W,), jnp.int32),
            pltpu.VMEM((BATCH, 8, ROWS_PER_W), jnp.float32),
            pltpu.VMEM((ROWS_PER_W,), jnp.float32),
            pltpu.SemaphoreType.DMA,
        ],
    )


def _tc_count_body(x_ref, v_ref, y_ref, out_ref, acc_ref):
    j = pl.program_id(1)

    @pl.when(j == 0)
    def _init():
        acc_ref[...] = jnp.zeros_like(acc_ref)

    x = x_ref[...]
    v = v_ref[...]
    yv = y_ref[...]
    cols = j * BCC + lax.broadcasted_iota(jnp.int32, (BCC, BRS), 0)
    m = (x > v) | ((x == v) & (cols < yv))
    ones = jnp.where(m, 1.0, 0.0)
    acc_ref[...] += ones.reshape(BCC // 8, 8, BRS).sum(axis=0)

    @pl.when(j == pl.num_programs(1) - 1)
    def _fin():
        rank = acc_ref[...].sum(axis=0, keepdims=True)
        match = jnp.where(rank < (TOPK - 0.5), 1.0, 0.0)
        part = jnp.sum(match, keepdims=True).reshape(1, 1)
        i = pl.program_id(0)
        prev = jnp.where(i == 0, jnp.zeros_like(part), out_ref[...])
        scale = jnp.where(i == pl.num_programs(0) - 1, 1.0 / NROWS, 1.0)
        out_ref[...] = (prev + part) * scale


def kernel(y_hat_logits, y):
    y32 = y.astype(jnp.int32)
    xt = y_hat_logits.T
    v1 = _sc_gather_kernel()(xt, y32)
    out = pl.pallas_call(
        _tc_count_body,
        grid=(NROWS // BRS, NCOLS // BCC),
        in_specs=[
            pl.BlockSpec((BCC, BRS), lambda i, j: (j, i)),
            pl.BlockSpec((1, BRS), lambda i, j: (0, i)),
            pl.BlockSpec((1, BRS), lambda i, j: (0, i)),
        ],
        out_specs=pl.BlockSpec((1, 1), lambda i, j: (0, 0)),
        out_shape=jax.ShapeDtypeStruct((1, 1), jnp.float32),
        scratch_shapes=[pltpu.VMEM((8, BRS), jnp.float32)],
    )(xt, v1.reshape(1, NROWS), y32.reshape(1, NROWS))
    return out[0, 0]


# speedup vs baseline: 13.1408x; 1.1009x over previous
import functools

import jax
import jax.numpy as jnp
from jax import lax
from jax.experimental import pallas as pl
from jax.experimental.pallas import tpu as pltpu
from jax.experimental.pallas import tpu_sc as plsc

TOPK = 5
NROWS = 4096
NCOLS = 100000
LANES = 16
NWORKERS = 32
ROWS_PER_W = NROWS // NWORKERS
BATCH = 64
BCC = 1000
BRS = 4096


def _sc_gather_body(xt_hbm, y_hbm, v_hbm, y_v, tiles_v, v_v, sem):
    wid = lax.axis_index("s") * 2 + lax.axis_index("c")
    base = pl.multiple_of(wid * ROWS_PER_W, ROWS_PER_W)
    pltpu.sync_copy(y_hbm.at[pl.ds(base, ROWS_PER_W)], y_v)
    iota = lax.iota(jnp.int32, LANES)
    for b in range(ROWS_PER_W // BATCH):
        for j in range(BATCH):
            lane = b * BATCH + j
            y_s = y_v[pl.ds((lane // LANES) * LANES, LANES)][lane % LANES]
            c8 = pl.multiple_of(y_s & ~7, 8)
            pltpu.async_copy(xt_hbm.at[pl.ds(c8, 8), pl.ds(base, ROWS_PER_W)],
                             tiles_v.at[j], sem)
        for j in range(BATCH):
            pltpu.make_async_copy(
                xt_hbm.at[pl.ds(0, 8), pl.ds(0, ROWS_PER_W)],
                tiles_v.at[j], sem).wait()
        for g in range(BATCH // LANES):
            grp = jnp.zeros((LANES,), jnp.float32)
            for k in range(LANES):
                j = g * LANES + k
                lane = b * BATCH + j
                y_s = y_v[pl.ds((lane // LANES) * LANES, LANES)][lane % LANES]
                s8 = y_s & 7
                vec = tiles_v[j, s8, pl.ds((lane // LANES) * LANES, LANES)]
                grp = jnp.where(iota == k, vec, grp)
            v_v[pl.ds(b * BATCH + g * LANES, LANES)] = grp
    pltpu.sync_copy(v_v, v_hbm.at[pl.ds(base, ROWS_PER_W)])


@functools.cache
def _sc_gather_kernel():
    return pl.kernel(
        _sc_gather_body,
        mesh=plsc.VectorSubcoreMesh(core_axis_name="c", subcore_axis_name="s"),
        out_type=jax.ShapeDtypeStruct((NROWS,), jnp.float32),
        scratch_types=[
            pltpu.VMEM((ROWS_PER_W,), jnp.int32),
            pltpu.VMEM((BATCH, 8, ROWS_PER_W), jnp.float32),
            pltpu.VMEM((ROWS_PER_W,), jnp.float32),
            pltpu.SemaphoreType.DMA,
        ],
    )


def _tc_count_body(x_ref, v_ref, y_ref, out_ref, acc_ref):
    j = pl.program_id(0)

    @pl.when(j == 0)
    def _init():
        acc_ref[...] = jnp.zeros_like(acc_ref)

    x = x_ref[...]
    v = v_ref[...]
    yv = y_ref[...]
    cols = j * BCC + lax.broadcasted_iota(jnp.int32, (BCC, BRS), 0)
    m = (x > v) | ((x == v) & (cols < yv))
    ones = jnp.where(m, 1.0, 0.0)
    acc_ref[...] += ones.reshape(BCC // 8, 8, BRS).sum(axis=0)

    @pl.when(j == pl.num_programs(0) - 1)
    def _fin():
        rank = acc_ref[...].sum(axis=0, keepdims=True)
        match = jnp.where(rank < (TOPK - 0.5), 1.0, 0.0)
        part = jnp.sum(match, keepdims=True).reshape(1, 1)
        out_ref[...] = part * (1.0 / NROWS)


def kernel(y_hat_logits, y):
    y32 = y.astype(jnp.int32)
    xt = y_hat_logits.T
    v1 = _sc_gather_kernel()(xt, y32)
    out = pl.pallas_call(
        _tc_count_body,
        grid=(NCOLS // BCC,),
        in_specs=[
            pl.BlockSpec((BCC, BRS), lambda j: (j, 0)),
            pl.BlockSpec((1, BRS), lambda j: (0, 0)),
            pl.BlockSpec((1, BRS), lambda j: (0, 0)),
        ],
        out_specs=pl.BlockSpec((1, 1), lambda j: (0, 0)),
        out_shape=jax.ShapeDtypeStruct((1, 1), jnp.float32),
        scratch_shapes=[pltpu.VMEM((8, BRS), jnp.float32)],
    )(xt, v1.reshape(1, NROWS), y32.reshape(1, NROWS))
    return out[0, 0]
